# baseline (speedup 1.0000x reference)
import jax
import jax.numpy as jnp
from jax import lax
from jax.experimental import pallas as pl
from jax.experimental.pallas import tpu as pltpu
from jax.experimental.pallas import tpu_sc as plsc

NC = 2
NS = 16
NW = NC * NS
L = 16

B_TOK = 4096
SEQ = 50
HID = 64
NVEC = HID // L
B = B_TOK * SEQ
BPW = B // NW
CH = 128
NCH = BPW // CH


def _splat_i32(v):
    return jnp.full((L,), v, dtype=jnp.int32)


def _group_normalize(buf, g):
    rows = g * L + lax.iota(jnp.int32, L)
    cols = [_splat_i32(k) for k in range(4)]
    four = _splat_i32(4)
    acc = [jnp.zeros((L,), jnp.float32) for _ in range(4)]
    for jj in range(HID // 4):
        for k in range(4):
            x = plsc.load_gather(buf, [rows, cols[k]])
            acc[k] = acc[k] + x * x
            if jj < HID // 4 - 1:
                cols[k] = cols[k] + four
    s = (acc[0] + acc[1]) + (acc[2] + acc[3])
    i = plsc.bitcast(s, jnp.int32)
    y = plsc.bitcast(jnp.int32(0x5F3759DF) - (i >> 1), jnp.float32)
    half = s * 0.5
    for _ in range(3):
        y = y * (1.5 - half * y * y)
    norm = s * y
    inv = 1.0 / (norm + 1e-8)
    cols = [_splat_i32(k) for k in range(4)]
    for jj in range(HID // 4):
        for k in range(4):
            x = plsc.load_gather(buf, [rows, cols[k]])
            plsc.store_scatter(buf, [rows, cols[k]], x * inv)
            if jj < HID // 4 - 1:
                cols[k] = cols[k] + four


def _body(idx_hbm, table_hbm, out_hbm, idx_v, buf, sem):
    wid = lax.axis_index("s") * NC + lax.axis_index("c")
    pltpu.sync_copy(idx_hbm.at[wid], idx_v)

    def chunk(j, carry):
        pltpu.async_copy(table_hbm.at[idx_v.at[j]], buf, sem).wait()

        def group(g, c):
            _group_normalize(buf, g)
            return c

        lax.fori_loop(0, CH // L, group, 0)
        pltpu.sync_copy(buf, out_hbm.at[pl.ds(wid * BPW + j * CH, CH)])
        return carry

    lax.fori_loop(0, NCH, chunk, 0)


@jax.jit
def _emb_call(idx, table):
    mesh = plsc.VectorSubcoreMesh(core_axis_name="c", subcore_axis_name="s")
    f = pl.kernel(
        _body,
        out_type=jax.ShapeDtypeStruct((B, HID), jnp.float32),
        mesh=mesh,
        compiler_params=pltpu.CompilerParams(
            needs_layout_passes=False, use_tc_tiling_on_sc=False
        ),
        scratch_types=[
            pltpu.VMEM((NCH, CH), jnp.int32),
            pltpu.VMEM((CH, HID), jnp.float32),
            pltpu.SemaphoreType.DMA,
        ],
    )
    return f(idx, table)


def kernel(input_ids, weight):
    idx = input_ids.reshape(NW, NCH, CH).astype(jnp.int32)
    out = _emb_call(idx, weight)
    return out.reshape(B_TOK, SEQ, HID)


# speedup vs baseline: 1.7583x; 1.7583x over previous
import jax
import jax.numpy as jnp
from jax import lax
from jax.experimental import pallas as pl
from jax.experimental.pallas import tpu as pltpu
from jax.experimental.pallas import tpu_sc as plsc

NC = 2
NS = 16
NW = NC * NS
L = 16

B_TOK = 4096
SEQ = 50
HID = 64
NVEC = HID // L
B = B_TOK * SEQ
BPW = B // NW
CH = 128
NCH = BPW // CH
NBUF = 5
NITER = NCH // NBUF

MAGIC = 0x5F3759DF


def _splat_i32(v):
    return jnp.full((L,), v, dtype=jnp.int32)


def _two_groups(buf, sq, g2):
    iota = lax.iota(jnp.int32, L)
    for h in range(2):
        base = (g2 * 2 + h) * L
        for r in range(L):
            v = [buf[base + r, pl.ds(p * L, L)] for p in range(NVEC)]
            s16 = (v[0] * v[0] + v[1] * v[1]) + (v[2] * v[2] + v[3] * v[3])
            sq[pl.ds(h * 256 + r * L, L)] = s16
        fbase = (iota << 4) + (h * 256)
        f = [fbase + kk for kk in range(4)]
        four = _splat_i32(4)
        acc = [None] * 4
        for step in range(4):
            for kk in range(4):
                x = plsc.load_gather(sq, [f[kk]])
                acc[kk] = x if step == 0 else acc[kk] + x
                if step < 3:
                    f[kk] = f[kk] + four
        s = (acc[0] + acc[1]) + (acc[2] + acc[3])
        iv = plsc.bitcast(s, jnp.int32)
        y = plsc.bitcast(jnp.full((L,), MAGIC, jnp.int32) - (iv >> 1), jnp.float32)
        half = s * 0.5
        for _ in range(3):
            y = y * (1.5 - half * y * y)
        inv = 1.0 / (s * y + 1e-8)
        for r in range(L):
            ivb = lax.broadcast_in_dim(inv[r], (L,), ())
            for p in range(NVEC):
                buf[base + r, pl.ds(p * L, L)] = buf[base + r, pl.ds(p * L, L)] * ivb


def _body(idx_hbm, table_hbm, out_hbm, idx_v, b0, b1, b2, b3, b4, sq,
          g0, g1, g2, g3, g4, w0, w1, w2, w3, w4):
    bufs = [b0, b1, b2, b3, b4]
    gsems = [g0, g1, g2, g3, g4]
    wsems = [w0, w1, w2, w3, w4]
    wid = lax.axis_index("s") * NC + lax.axis_index("c")
    pltpu.sync_copy(idx_hbm.at[wid], idx_v)

    for k in range(NBUF - 1):
        pltpu.async_copy(table_hbm.at[idx_v.at[k]], bufs[k], gsems[k])

    def iter_body(t, carry):
        for k in range(NBUF):
            c = t * NBUF + k
            pltpu.make_async_copy(
                table_hbm.at[idx_v.at[c]], bufs[k], gsems[k]
            ).wait()

            def pair(g2i, cc, _buf=bufs[k]):
                _two_groups(_buf, sq, g2i)
                return cc

            lax.fori_loop(0, CH // (2 * L), pair, 0)
            pltpu.async_copy(
                bufs[k], out_hbm.at[pl.ds(wid * BPW + c * CH, CH)], wsems[k]
            )
            n = c + NBUF - 1
            ps = (k + NBUF - 1) % NBUF

            @pl.when(n < NCH)
            def _(k=k, c=c, n=n, ps=ps):
                @pl.when(c >= 1)
                def _():
                    pltpu.make_async_copy(
                        bufs[ps],
                        out_hbm.at[pl.ds(wid * BPW + (c - 1) * CH, CH)],
                        wsems[ps],
                    ).wait()

                pltpu.async_copy(table_hbm.at[idx_v.at[n]], bufs[ps], gsems[ps])

        return carry

    lax.fori_loop(0, NITER, iter_body, 0)
    for k in range(NBUF):
        c = NCH - NBUF + k
        pltpu.make_async_copy(
            bufs[k], out_hbm.at[pl.ds(wid * BPW + c * CH, CH)], wsems[k]
        ).wait()


@jax.jit
def _emb_call(idx, table):
    mesh = plsc.VectorSubcoreMesh(core_axis_name="c", subcore_axis_name="s")
    f = pl.kernel(
        _body,
        out_type=jax.ShapeDtypeStruct((B, HID), jnp.float32),
        mesh=mesh,
        compiler_params=pltpu.CompilerParams(
            needs_layout_passes=False, use_tc_tiling_on_sc=False
        ),
        scratch_types=(
            [pltpu.VMEM((NCH, CH), jnp.int32)]
            + [pltpu.VMEM((CH, HID), jnp.float32) for _ in range(NBUF)]
            + [pltpu.VMEM((2 * 256,), jnp.float32)]
            + [pltpu.SemaphoreType.DMA for _ in range(2 * NBUF)]
        ),
    )
    return f(idx, table)


def kernel(input_ids, weight):
    idx = input_ids.reshape(NW, NCH, CH).astype(jnp.int32)
    out = _emb_call(idx, weight)
    return out.reshape(B_TOK, SEQ, HID)


# speedup vs baseline: 1.8126x; 1.0309x over previous
import jax
import jax.numpy as jnp
from jax import lax
from jax.experimental import pallas as pl
from jax.experimental.pallas import tpu as pltpu
from jax.experimental.pallas import tpu_sc as plsc

NC = 2
NS = 16
NW = NC * NS
L = 16

B_TOK = 4096
SEQ = 50
HID = 64
NVEC = HID // L
B = B_TOK * SEQ
BPW = B // NW
HIDP = 128
CH = 128
NCH = BPW // CH
NBUF = 5
NITER = NCH // NBUF

MAGIC = 0x5F3759DF


def _splat_i32(v):
    return jnp.full((L,), v, dtype=jnp.int32)


def _two_groups(buf, sq, g2):
    iota = lax.iota(jnp.int32, L)
    for h in range(2):
        base = (g2 * 2 + h) * L
        for r in range(L):
            v = [buf[base + r, pl.ds(p * L, L)] for p in range(NVEC)]
            s16 = (v[0] * v[0] + v[1] * v[1]) + (v[2] * v[2] + v[3] * v[3])
            sq[pl.ds(h * 256 + r * L, L)] = s16
        fbase = (iota << 4) + (h * 256)
        f = [fbase + kk for kk in range(4)]
        four = _splat_i32(4)
        acc = [None] * 4
        for step in range(4):
            for kk in range(4):
                x = plsc.load_gather(sq, [f[kk]])
                acc[kk] = x if step == 0 else acc[kk] + x
                if step < 3:
                    f[kk] = f[kk] + four
        s = (acc[0] + acc[1]) + (acc[2] + acc[3])
        iv = plsc.bitcast(s, jnp.int32)
        y = plsc.bitcast(jnp.full((L,), MAGIC, jnp.int32) - (iv >> 1), jnp.float32)
        half = s * 0.5
        for _ in range(3):
            y = y * (1.5 - half * y * y)
        inv = 1.0 / (s * y + 1e-8)
        for r in range(L):
            ivb = lax.broadcast_in_dim(inv[r], (L,), ())
            for p in range(NVEC):
                buf[base + r, pl.ds(p * L, L)] = buf[base + r, pl.ds(p * L, L)] * ivb


def _body(idx_hbm, table_hbm, out_hbm, idx_v, b0, b1, b2, b3, b4, sq,
          g0, g1, g2, g3, g4, w0, w1, w2, w3, w4):
    bufs = [b0, b1, b2, b3, b4]
    gsems = [g0, g1, g2, g3, g4]
    wsems = [w0, w1, w2, w3, w4]
    wid = lax.axis_index("s") * NC + lax.axis_index("c")
    pltpu.sync_copy(idx_hbm.at[wid], idx_v)

    for k in range(NBUF - 1):
        pltpu.async_copy(table_hbm.at[idx_v.at[k]], bufs[k], gsems[k])

    def iter_body(t, carry):
        for k in range(NBUF):
            c = t * NBUF + k
            pltpu.make_async_copy(
                table_hbm.at[idx_v.at[c]], bufs[k], gsems[k]
            ).wait()

            def pair(g2i, cc, _buf=bufs[k]):
                _two_groups(_buf, sq, g2i)
                return cc

            lax.fori_loop(0, CH // (2 * L), pair, 0)
            pltpu.async_copy(
                bufs[k].at[:, pl.ds(0, HID)],
                out_hbm.at[pl.ds(wid * BPW + c * CH, CH)],
                wsems[k],
            )
            n = c + NBUF - 1
            ps = (k + NBUF - 1) % NBUF

            @pl.when(n < NCH)
            def _(k=k, c=c, n=n, ps=ps):
                @pl.when(c >= 1)
                def _():
                    pltpu.make_async_copy(
                        bufs[ps],
                        out_hbm.at[pl.ds(wid * BPW + (c - 1) * CH, CH)],
                        wsems[ps],
                    ).wait()

                pltpu.async_copy(table_hbm.at[idx_v.at[n]], bufs[ps], gsems[ps])

        return carry

    lax.fori_loop(0, NITER, iter_body, 0)
    for k in range(NBUF):
        c = NCH - NBUF + k
        pltpu.make_async_copy(
            bufs[k].at[:, pl.ds(0, HID)],
            out_hbm.at[pl.ds(wid * BPW + c * CH, CH)],
            wsems[k],
        ).wait()


@jax.jit
def _emb_call(idx, table):
    mesh = plsc.VectorSubcoreMesh(core_axis_name="c", subcore_axis_name="s")
    f = pl.kernel(
        _body,
        out_type=jax.ShapeDtypeStruct((B, HID), jnp.float32),
        mesh=mesh,
        compiler_params=pltpu.CompilerParams(
            needs_layout_passes=False, use_tc_tiling_on_sc=False
        ),
        scratch_types=(
            [pltpu.VMEM((NCH, CH), jnp.int32)]
            + [pltpu.VMEM((CH, HIDP), jnp.float32) for _ in range(NBUF)]
            + [pltpu.VMEM((2 * 256,), jnp.float32)]
            + [pltpu.SemaphoreType.DMA for _ in range(2 * NBUF)]
        ),
    )
    return f(idx, table)


def kernel(input_ids, weight):
    idx = input_ids.reshape(NW, NCH, CH).astype(jnp.int32)
    wp = jnp.pad(weight, ((0, 0), (0, HIDP - HID)))
    out = _emb_call(idx, wp)
    return out.reshape(B_TOK, SEQ, HID)
